# baseline (speedup 1.0000x reference)
import functools

import jax
import jax.numpy as jnp
from jax.experimental import pallas as pl
from jax.experimental.pallas import tpu as pltpu


def _sage_kernel(a_ref, h_ref, wt_ref, o_ref, *, row_blk):
    i = pl.program_id(0)
    a = a_ref[...]
    s = jax.lax.dot_general(
        a, h_ref[...], (((1,), (0,)), ((), ())),
        precision=jax.lax.Precision.DEFAULT,
        preferred_element_type=jnp.float32,
    )
    deg = jnp.sum(a, axis=1, keepdims=True)
    hr = h_ref[pl.ds(i * row_blk, row_blk), :]
    agg = (s + hr) / (deg + 1.0)
    z = jnp.dot(agg, wt_ref[...], preferred_element_type=jnp.float32)
    o_ref[...] = jnp.where(z >= 0.0, z, 0.01 * z)


def _pick_block(n, target):
    if n % target == 0:
        return target
    return n


def kernel(A, h, weight):
    n, d = h.shape
    row_blk = _pick_block(n, 200)
    wt = weight.T

    out = pl.pallas_call(
        functools.partial(_sage_kernel, row_blk=row_blk),
        grid=(n // row_blk,),
        in_specs=[
            pl.BlockSpec((row_blk, n), lambda i: (i, 0)),
            pl.BlockSpec((n, d), lambda i: (0, 0)),
            pl.BlockSpec((d, d), lambda i: (0, 0)),
        ],
        out_specs=pl.BlockSpec((row_blk, d), lambda i: (i, 0)),
        out_shape=jax.ShapeDtypeStruct((n, d), jnp.float32),
        compiler_params=pltpu.CompilerParams(
            dimension_semantics=("arbitrary",),
        ),
    )(A, h, wt)
    return out


# speedup vs baseline: 1.0414x; 1.0414x over previous
import functools

import jax
import jax.numpy as jnp
from jax.experimental import pallas as pl
from jax.experimental.pallas import tpu as pltpu


def _sage_kernel(a_ref, h_ref, wt_ref, o_ref, *, row_blk):
    i = pl.program_id(0)
    a = a_ref[...]
    s = jax.lax.dot_general(
        a, h_ref[...], (((1,), (0,)), ((), ())),
        precision=jax.lax.Precision.DEFAULT,
        preferred_element_type=jnp.float32,
    )
    deg = jnp.sum(a, axis=1, keepdims=True)
    hr = h_ref[pl.ds(i * row_blk, row_blk), :]
    agg = (s + hr) / (deg + 1.0)
    z = jnp.dot(agg, wt_ref[...], preferred_element_type=jnp.float32)
    o_ref[...] = jnp.where(z >= 0.0, z, 0.01 * z)


def _pick_block(n, target):
    if n % target == 0:
        return target
    return n


def kernel(A, h, weight):
    n, d = h.shape
    row_blk = _pick_block(n, 400)
    wt = weight.T

    out = pl.pallas_call(
        functools.partial(_sage_kernel, row_blk=row_blk),
        grid=(n // row_blk,),
        in_specs=[
            pl.BlockSpec((row_blk, n), lambda i: (i, 0)),
            pl.BlockSpec((n, d), lambda i: (0, 0)),
            pl.BlockSpec((d, d), lambda i: (0, 0)),
        ],
        out_specs=pl.BlockSpec((row_blk, d), lambda i: (i, 0)),
        out_shape=jax.ShapeDtypeStruct((n, d), jnp.float32),
        compiler_params=pltpu.CompilerParams(
            dimension_semantics=("arbitrary",),
        ),
    )(A, h, wt)
    return out
